# baseline (speedup 1.0000x reference)
import jax
import jax.numpy as jnp
from jax.experimental import pallas as pl
from jax.experimental.pallas import tpu as pltpu

_N = 10000
_IN_F = 128
_MID = 32
_OUT_F = 128
_BM = 200
_STEPS = _N // (2 * _BM)


def _gcn_kernel(x_ref, w1_ref, adj_a_ref, adj_b_ref, w2_ref, out_ref, hid_ref):
    @pl.when(pl.program_id(0) == 0)
    def _():
        h = jnp.dot(
            x_ref[...].astype(jnp.bfloat16),
            w1_ref[...].astype(jnp.bfloat16),
            preferred_element_type=jnp.float32,
        )
        hid_ref[...] = h.astype(jnp.bfloat16)

    w2 = w2_ref[...].astype(jnp.bfloat16)
    t_a = jnp.dot(
        adj_a_ref[...].astype(jnp.bfloat16),
        hid_ref[...],
        preferred_element_type=jnp.float32,
    )
    out_ref[0:_BM, :] = jnp.dot(
        t_a.astype(jnp.bfloat16), w2, preferred_element_type=jnp.float32
    )
    t_b = jnp.dot(
        adj_b_ref[...].astype(jnp.bfloat16),
        hid_ref[...],
        preferred_element_type=jnp.float32,
    )
    out_ref[_BM:2 * _BM, :] = jnp.dot(
        t_b.astype(jnp.bfloat16), w2, preferred_element_type=jnp.float32
    )


def kernel(input, adj, weight1, weight2):
    return pl.pallas_call(
        _gcn_kernel,
        grid=(_STEPS,),
        in_specs=[
            pl.BlockSpec((_N, _IN_F), lambda i: (0, 0)),
            pl.BlockSpec((_IN_F, _MID), lambda i: (0, 0)),
            pl.BlockSpec((_BM, _N), lambda i: (2 * i, 0)),
            pl.BlockSpec((_BM, _N), lambda i: (2 * i + 1, 0)),
            pl.BlockSpec((_MID, _OUT_F), lambda i: (0, 0)),
        ],
        out_specs=pl.BlockSpec((2 * _BM, _OUT_F), lambda i: (i, 0)),
        out_shape=jax.ShapeDtypeStruct((_N, _OUT_F), jnp.float32),
        scratch_shapes=[pltpu.VMEM((_N, _MID), jnp.bfloat16)],
        compiler_params=pltpu.CompilerParams(
            dimension_semantics=("arbitrary",),
        ),
    )(input, weight1, adj, adj, weight2)
